# Initial kernel scaffold; baseline (speedup 1.0000x reference)
import jax
import jax.numpy as jnp
from jax.experimental import pallas as pl


def kernel(x, edge_index, batch_seg, W1, b1, W2, b2, W_out, b_out):
    raise NotImplementedError("write your pallas kernel here")


# speedup vs baseline: 12.9784x; 12.9784x over previous
import functools

import jax
import jax.numpy as jnp
from jax import lax
from jax.experimental import pallas as pl
from jax.experimental.pallas import tpu as pltpu
from jax.experimental.pallas import tpu_sc as plsc

NC = 2
NS = 16
NW = NC * NS


def _deg_body(nchunk, chunk, rt, dst_hbm, ones_hbm, zeros_hbm, out_hbm,
              idx_v, ones_v, cnt_sh):
    c = lax.axis_index("c")
    s = lax.axis_index("s")
    wid = s * NC + c
    ew = nchunk * chunk
    pltpu.sync_copy(ones_hbm, ones_v)
    pltpu.sync_copy(zeros_hbm.at[pl.ds(s * rt, rt)], cnt_sh.at[pl.ds(s * rt, rt)])
    plsc.subcore_barrier()

    def body(g, carry):
        base = pl.multiple_of(wid * ew + g * chunk, 8)
        pltpu.sync_copy(dst_hbm.at[pl.ds(base, chunk)], idx_v)
        pltpu.sync_copy(ones_v, cnt_sh.at[idx_v], add=True)
        return carry

    lax.fori_loop(0, nchunk, body, 0)
    plsc.subcore_barrier()
    pltpu.sync_copy(cnt_sh.at[pl.ds(s * rt, rt)], out_hbm.at[c, pl.ds(s * rt, rt)])


def _agg_body(nchunk, chunk, rt, hp_hbm, src_hbm, dst_hbm, zeros_hbm, out_hbm,
              sidx_v, didx_v, rows_v, sem, acc_sh):
    c = lax.axis_index("c")
    s = lax.axis_index("s")
    wid = s * NC + c
    ew = nchunk * chunk
    pltpu.sync_copy(zeros_hbm.at[pl.ds(s * rt, rt)], acc_sh.at[pl.ds(s * rt, rt)])
    plsc.subcore_barrier()

    def body(g, carry):
        base = pl.multiple_of(wid * ew + g * chunk, 8)
        pltpu.sync_copy(src_hbm.at[pl.ds(base, chunk)], sidx_v)
        pltpu.sync_copy(dst_hbm.at[pl.ds(base, chunk)], didx_v)
        pltpu.async_copy(hp_hbm.at[sidx_v], rows_v, sem).wait()
        pltpu.sync_copy(rows_v, acc_sh.at[didx_v], add=True)
        return carry

    lax.fori_loop(0, nchunk, body, 0)
    plsc.subcore_barrier()
    pltpu.sync_copy(acc_sh.at[pl.ds(s * rt, rt)], out_hbm.at[c, pl.ds(s * rt, rt)])


def _pad_rows(n):
    return -(-n // (NS * 8)) * (NS * 8)


def _sc_deg(dst, n, chunk=80):
    e = dst.shape[0]
    ew = e // NW
    nchunk = ew // chunk
    npad = _pad_rows(n)
    rt = npad // NS
    mesh = plsc.VectorSubcoreMesh(core_axis_name="c", subcore_axis_name="s")
    ones = jnp.ones((chunk, 16), jnp.float32)
    zeros = jnp.zeros((npad, 16), jnp.float32)
    f = functools.partial(
        pl.kernel,
        out_type=jax.ShapeDtypeStruct((NC, npad, 16), jnp.float32),
        mesh=mesh,
        scratch_types=[
            pltpu.VMEM((chunk,), jnp.int32),
            pltpu.VMEM((chunk, 16), jnp.float32),
            pltpu.VMEM_SHARED((npad, 16), jnp.float32),
        ],
    )(functools.partial(_deg_body, nchunk, chunk, rt))
    return f(dst, ones, zeros)


def _sc_agg(hp, src, dst, zeros_nd, chunk=80):
    n, d = hp.shape
    e = src.shape[0]
    ew = e // NW
    nchunk = ew // chunk
    npad = _pad_rows(n)
    rt = npad // NS
    mesh = plsc.VectorSubcoreMesh(core_axis_name="c", subcore_axis_name="s")
    f = functools.partial(
        pl.kernel,
        out_type=jax.ShapeDtypeStruct((NC, npad, d), jnp.float32),
        mesh=mesh,
        scratch_types=[
            pltpu.VMEM((chunk,), jnp.int32),
            pltpu.VMEM((chunk,), jnp.int32),
            pltpu.VMEM((chunk, d), jnp.float32),
            pltpu.SemaphoreType.DMA,
            pltpu.VMEM_SHARED((npad, d), jnp.float32),
        ],
    )(functools.partial(_agg_body, nchunk, chunk, rt))
    return f(hp, src, dst, zeros_nd)


def _dis_of(cnt_ref, n):
    cnt = cnt_ref[0, :n] + cnt_ref[1, :n]
    return lax.rsqrt(cnt + 1.0)[:, 0:1]


def _l1_body(x_ref, w_ref, cnt_ref, hp_ref):
    dis = _dis_of(cnt_ref, x_ref.shape[0])
    hp_ref[...] = jnp.dot(x_ref[...], w_ref[...],
                          preferred_element_type=jnp.float32) * dis


def _l2_body(parts_ref, hp1_ref, cnt_ref, b1_ref, w2_ref, hp2_ref):
    n = hp1_ref.shape[0]
    dis = _dis_of(cnt_ref, n)
    h1 = dis * (parts_ref[0, :n] + parts_ref[1, :n] + hp1_ref[...]) + b1_ref[...]
    h1 = jnp.maximum(h1, 0.0)
    hp2_ref[...] = jnp.dot(h1, w2_ref[...],
                           preferred_element_type=jnp.float32) * dis


def _fin_body(parts_ref, hp2_ref, cnt_ref, b2_ref, seg_ref, wout_ref, bout_ref,
              out_ref):
    g = out_ref.shape[0]
    n = hp2_ref.shape[0]
    dis = _dis_of(cnt_ref, n)
    h2 = dis * (parts_ref[0, :n] + parts_ref[1, :n] + hp2_ref[...]) + b2_ref[...]
    gids = lax.broadcasted_iota(jnp.int32, (g, n), 0)
    onehot = (gids == seg_ref[...]).astype(jnp.float32)
    sums = jnp.dot(onehot, h2, preferred_element_type=jnp.float32)
    cnts = jnp.sum(onehot, axis=1, keepdims=True)
    pooled = sums / jnp.maximum(cnts, 1.0)
    out_ref[...] = jnp.dot(pooled, wout_ref[...],
                           preferred_element_type=jnp.float32) + bout_ref[...]


def _tc(body, out_shape):
    return pl.pallas_call(body, out_shape=out_shape)


def kernel(x, edge_index, batch_seg, W1, b1, W2, b2, W_out, b_out):
    n, d = x.shape
    g = W_out.shape[1]
    src = edge_index[0]
    dst = edge_index[1]
    zeros_nd = jnp.zeros((_pad_rows(n), d), jnp.float32)

    cnt_parts = _sc_deg(dst, n)

    hp1 = _tc(_l1_body, jax.ShapeDtypeStruct((n, d), jnp.float32))(
        x, W1, cnt_parts)
    parts1 = _sc_agg(hp1, src, dst, zeros_nd)

    hp2 = _tc(_l2_body, jax.ShapeDtypeStruct((n, d), jnp.float32))(
        parts1, hp1, cnt_parts, b1, W2)
    parts2 = _sc_agg(hp2, src, dst, zeros_nd)

    seg2d = batch_seg.reshape(1, n)
    ng = 64
    out = _tc(_fin_body, jax.ShapeDtypeStruct((ng, W_out.shape[1]), jnp.float32))(
        parts2, hp2, cnt_parts, b2, seg2d, W_out, b_out)
    return out


# speedup vs baseline: 19.1657x; 1.4767x over previous
import functools

import jax
import jax.numpy as jnp
from jax import lax
from jax.experimental import pallas as pl
from jax.experimental.pallas import tpu as pltpu
from jax.experimental.pallas import tpu_sc as plsc

NC = 2
NS = 16
NW = NC * NS


def _row_split(n):
    rtb = (n // (NS * 8)) * 8
    tail = n - rtb * NS
    return rtb, tail


def _copy_rows(src, dst, s, rtb, tail, ns_base):
    pltpu.sync_copy(src.at[pl.ds(s * rtb, rtb)], dst.at[pl.ds(s * rtb, rtb)])
    if tail:
        @pl.when(s == NS - 1)
        def _():
            pltpu.sync_copy(src.at[pl.ds(ns_base, tail)],
                            dst.at[pl.ds(ns_base, tail)])


def _deg_body(n, nchunk, chunk, ring, dst_hbm, ones_hbm, zeros_hbm, out_hbm,
              *rest):
    idx = rest[0:ring]
    ones_v = rest[ring]
    sem_i = rest[ring + 1:2 * ring + 1]
    cnt_sh = rest[2 * ring + 1]
    rtb, tail = _row_split(n)
    c = lax.axis_index("c")
    s = lax.axis_index("s")
    wid = s * NC + c
    ebase = wid * (nchunk * chunk)

    def idx_issue(g, q):
        off = pl.multiple_of(ebase + g * chunk, 8)
        pltpu.async_copy(dst_hbm.at[pl.ds(off, chunk)], idx[q], sem_i[q])

    def idx_wait(g, q):
        off = pl.multiple_of(ebase + g * chunk, 8)
        pltpu.make_async_copy(dst_hbm.at[pl.ds(off, chunk)], idx[q],
                              sem_i[q]).wait()

    pltpu.sync_copy(ones_hbm, ones_v)
    for q in range(ring):
        idx_issue(q, q)
    _copy_rows(zeros_hbm, cnt_sh, s, rtb, tail, NS * rtb)
    plsc.subcore_barrier()

    def outer(i, carry):
        for b in range(ring):
            g = i * ring + b
            idx_wait(g, b)
            pltpu.sync_copy(ones_v, cnt_sh.at[idx[b]], add=True)

            @pl.when(g + ring < nchunk)
            def _():
                idx_issue(g + ring, b)
        return carry

    lax.fori_loop(0, nchunk // ring, outer, 0)
    plsc.subcore_barrier()
    _copy_rows(cnt_sh, out_hbm.at[c], s, rtb, tail, NS * rtb)


def _agg_body(n, nchunk, chunk, ring, hp_hbm, src_hbm, dst_hbm, zeros_hbm,
              out_hbm, rows_v, *rest):
    k = ring - 1
    sidx_v = rest[0:ring]
    didx_v = rest[ring:2 * ring]
    rest = rest[2 * ring:]
    sem_si = rest[0:ring]
    sem_di = rest[ring:2 * ring]
    sem_g = rest[2 * ring:3 * ring]
    acc_sh = rest[3 * ring]
    rtb, tail = _row_split(n)
    c = lax.axis_index("c")
    s = lax.axis_index("s")
    wid = s * NC + c
    ebase = wid * (nchunk * chunk)

    def idx_issue(g, q):
        off = pl.multiple_of(ebase + g * chunk, 8)
        pltpu.async_copy(src_hbm.at[pl.ds(off, chunk)], sidx_v[q], sem_si[q])
        pltpu.async_copy(dst_hbm.at[pl.ds(off, chunk)], didx_v[q], sem_di[q])

    def si_wait(g, q):
        off = pl.multiple_of(ebase + g * chunk, 8)
        pltpu.make_async_copy(src_hbm.at[pl.ds(off, chunk)], sidx_v[q],
                              sem_si[q]).wait()

    def di_wait(g, q):
        off = pl.multiple_of(ebase + g * chunk, 8)
        pltpu.make_async_copy(dst_hbm.at[pl.ds(off, chunk)], didx_v[q],
                              sem_di[q]).wait()

    def gather_issue(q):
        pltpu.async_copy(hp_hbm.at[sidx_v[q]], rows_v.at[q], sem_g[q])

    def gather_wait(q):
        pltpu.make_async_copy(hp_hbm.at[sidx_v[q]], rows_v.at[q],
                              sem_g[q]).wait()

    for q in range(ring):
        idx_issue(q, q)
    _copy_rows(zeros_hbm, acc_sh, s, rtb, tail, NS * rtb)
    plsc.subcore_barrier()
    for g in range(k):
        si_wait(g, g)
        gather_issue(g)

    def outer(i, carry):
        for b in range(ring):
            g = i * ring + b
            qk = (b + k) % ring

            @pl.when(g + k < nchunk)
            def _():
                si_wait(g + k, qk)
                gather_issue(qk)

            gather_wait(b)
            di_wait(g, b)
            pltpu.sync_copy(rows_v.at[b], acc_sh.at[didx_v[b]], add=True)

            @pl.when(g + ring < nchunk)
            def _():
                idx_issue(g + ring, b)
        return carry

    lax.fori_loop(0, nchunk // ring, outer, 0)
    plsc.subcore_barrier()
    _copy_rows(acc_sh, out_hbm.at[c], s, rtb, tail, NS * rtb)


def _sc_deg(dst, n, d, zeros_nd, chunk=40, ring=5):
    e = dst.shape[0]
    nchunk = e // (NW * chunk)
    mesh = plsc.VectorSubcoreMesh(core_axis_name="c", subcore_axis_name="s")
    ones = jnp.ones((chunk, d), jnp.float32)
    f = functools.partial(
        pl.kernel,
        out_type=jax.ShapeDtypeStruct((NC, n, d), jnp.float32),
        mesh=mesh,
        scratch_types=(
            [pltpu.VMEM((chunk,), jnp.int32)] * ring
            + [pltpu.VMEM((chunk, d), jnp.float32)]
            + [pltpu.SemaphoreType.DMA] * ring
            + [pltpu.VMEM_SHARED((n, d), jnp.float32)]
        ),
    )(functools.partial(_deg_body, n, nchunk, chunk, ring))
    return f(dst, ones, zeros_nd)


def _sc_agg(hp, src, dst, zeros_nd, chunk=40, ring=5):
    n, d = hp.shape
    e = src.shape[0]
    nchunk = e // (NW * chunk)
    mesh = plsc.VectorSubcoreMesh(core_axis_name="c", subcore_axis_name="s")
    f = functools.partial(
        pl.kernel,
        out_type=jax.ShapeDtypeStruct((NC, n, d), jnp.float32),
        mesh=mesh,
        scratch_types=(
            [pltpu.VMEM((ring, chunk, d), jnp.float32)]
            + [pltpu.VMEM((chunk,), jnp.int32)] * (2 * ring)
            + [pltpu.SemaphoreType.DMA] * (3 * ring)
            + [pltpu.VMEM_SHARED((n, d), jnp.float32)]
        ),
    )(functools.partial(_agg_body, n, nchunk, chunk, ring))
    return f(hp, src, dst, zeros_nd)


def _dis_of(cnt_ref):
    cnt = cnt_ref[0] + cnt_ref[1]
    return lax.rsqrt(cnt + 1.0)[:, 0:1]


def _l1_body(x_ref, w_ref, cnt_ref, hp_ref):
    dis = _dis_of(cnt_ref)
    hp_ref[...] = jnp.dot(x_ref[...], w_ref[...],
                          preferred_element_type=jnp.float32) * dis


def _l2_body(parts_ref, hp1_ref, cnt_ref, b1_ref, w2_ref, hp2_ref):
    dis = _dis_of(cnt_ref)
    h1 = dis * (parts_ref[0] + parts_ref[1] + hp1_ref[...]) + b1_ref[...]
    h1 = jnp.maximum(h1, 0.0)
    hp2_ref[...] = jnp.dot(h1, w2_ref[...],
                           preferred_element_type=jnp.float32) * dis


def _fin_body(parts_ref, hp2_ref, cnt_ref, b2_ref, seg_ref, wout_ref, bout_ref,
              out_ref):
    g = out_ref.shape[0]
    n = hp2_ref.shape[0]
    dis = _dis_of(cnt_ref)
    h2 = dis * (parts_ref[0] + parts_ref[1] + hp2_ref[...]) + b2_ref[...]
    gids = lax.broadcasted_iota(jnp.int32, (g, n), 0)
    onehot = (gids == seg_ref[...]).astype(jnp.float32)
    sums = jnp.dot(onehot, h2, preferred_element_type=jnp.float32)
    cnts = jnp.sum(onehot, axis=1, keepdims=True)
    pooled = sums / jnp.maximum(cnts, 1.0)
    out_ref[...] = jnp.dot(pooled, wout_ref[...],
                           preferred_element_type=jnp.float32) + bout_ref[...]


def _tc(body, out_shape):
    return pl.pallas_call(body, out_shape=out_shape)


def kernel(x, edge_index, batch_seg, W1, b1, W2, b2, W_out, b_out):
    n, d = x.shape
    chunk = 40
    e = edge_index.shape[1]
    nchunk = e // (NW * chunk)
    src = edge_index[0]
    dst = edge_index[1]
    zeros_nd = jnp.zeros((n, d), jnp.float32)

    cnt_parts = _sc_deg(dst, n, d, zeros_nd, chunk)

    hp1 = _tc(_l1_body, jax.ShapeDtypeStruct((n, d), jnp.float32))(
        x, W1, cnt_parts)
    parts1 = _sc_agg(hp1, src, dst, zeros_nd, chunk)

    hp2 = _tc(_l2_body, jax.ShapeDtypeStruct((n, d), jnp.float32))(
        parts1, hp1, cnt_parts, b1, W2)
    parts2 = _sc_agg(hp2, src, dst, zeros_nd, chunk)

    seg2d = batch_seg.reshape(1, n)
    ng = 64
    out = _tc(_fin_body, jax.ShapeDtypeStruct((ng, W_out.shape[1]), jnp.float32))(
        parts2, hp2, cnt_parts, b2, seg2d, W_out, b_out)
    return out


# speedup vs baseline: 29.8478x; 1.5574x over previous
import functools

import jax
import jax.numpy as jnp
from jax import lax
from jax.experimental import pallas as pl
from jax.experimental.pallas import tpu as pltpu
from jax.experimental.pallas import tpu_sc as plsc

NC = 2
NS = 16
NW = NC * NS


def _row_split(n):
    rtb = (n // (NS * 8)) * 8
    tail = n - rtb * NS
    return rtb, tail


def _copy_rows(src, dst, s, rtb, tail, ns_base):
    pltpu.sync_copy(src.at[pl.ds(s * rtb, rtb)], dst.at[pl.ds(s * rtb, rtb)])
    if tail:
        @pl.when(s == NS - 1)
        def _():
            pltpu.sync_copy(src.at[pl.ds(ns_base, tail)],
                            dst.at[pl.ds(ns_base, tail)])


def _deg_body(n, nchunk, chunk, ring, dst_hbm, ones_hbm, zeros_hbm, out_hbm,
              *rest):
    idx = rest[0:ring]
    ones_v = rest[ring]
    sem_i = rest[ring + 1:2 * ring + 1]
    cnt_sh = rest[2 * ring + 1]
    rtb, tail = _row_split(n)
    c = lax.axis_index("c")
    s = lax.axis_index("s")
    wid = s * NC + c
    ebase = wid * (nchunk * chunk)

    def idx_issue(g, q):
        off = pl.multiple_of(ebase + g * chunk, 8)
        pltpu.async_copy(dst_hbm.at[pl.ds(off, chunk)], idx[q], sem_i[q])

    def idx_wait(g, q):
        off = pl.multiple_of(ebase + g * chunk, 8)
        pltpu.make_async_copy(dst_hbm.at[pl.ds(off, chunk)], idx[q],
                              sem_i[q]).wait()

    pltpu.sync_copy(ones_hbm, ones_v)
    for q in range(ring):
        idx_issue(q, q)
    _copy_rows(zeros_hbm, cnt_sh, s, rtb, tail, NS * rtb)
    plsc.subcore_barrier()

    def outer(i, carry):
        for b in range(ring):
            g = i * ring + b
            idx_wait(g, b)
            pltpu.sync_copy(ones_v, cnt_sh.at[idx[b]], add=True)

            @pl.when(g + ring < nchunk)
            def _():
                idx_issue(g + ring, b)
        return carry

    lax.fori_loop(0, nchunk // ring, outer, 0)
    plsc.subcore_barrier()
    _copy_rows(cnt_sh, out_hbm.at[c], s, rtb, tail, NS * rtb)


def _agg_body(n, nchunk, chunk, ring, hp_hbm, src_hbm, dst_hbm, zeros_hbm,
              out_hbm, rows_v, *rest):
    k = ring - 1
    nq = 2 * ring
    sidx_v = rest[0:nq]
    didx_v = rest[nq:2 * nq]
    rest = rest[2 * nq:]
    sem_si = rest[0:nq]
    sem_di = rest[nq:2 * nq]
    sem_g = rest[2 * nq:2 * nq + ring]
    sem_sc = rest[2 * nq + ring:2 * nq + 2 * ring]
    acc_sh = rest[2 * nq + 2 * ring]
    rtb, tail = _row_split(n)
    c = lax.axis_index("c")
    s = lax.axis_index("s")
    wid = s * NC + c
    ebase = wid * (nchunk * chunk)

    def idx_issue(g, q):
        off = pl.multiple_of(ebase + g * chunk, 8)
        pltpu.async_copy(src_hbm.at[pl.ds(off, chunk)], sidx_v[q], sem_si[q])
        pltpu.async_copy(dst_hbm.at[pl.ds(off, chunk)], didx_v[q], sem_di[q])

    def si_wait(g, q):
        off = pl.multiple_of(ebase + g * chunk, 8)
        pltpu.make_async_copy(src_hbm.at[pl.ds(off, chunk)], sidx_v[q],
                              sem_si[q]).wait()

    def di_wait(g, q):
        off = pl.multiple_of(ebase + g * chunk, 8)
        pltpu.make_async_copy(dst_hbm.at[pl.ds(off, chunk)], didx_v[q],
                              sem_di[q]).wait()

    def gather_issue(qi, qr):
        pltpu.async_copy(hp_hbm.at[sidx_v[qi]], rows_v.at[qr], sem_g[qr])

    def gather_wait(qi, qr):
        pltpu.make_async_copy(hp_hbm.at[sidx_v[qi]], rows_v.at[qr],
                              sem_g[qr]).wait()

    def scatter_issue(qi, qr):
        pltpu.async_copy(rows_v.at[qr], acc_sh.at[didx_v[qi]], sem_sc[qr],
                         add=True)

    def scatter_wait(qi, qr):
        pltpu.make_async_copy(rows_v.at[qr], acc_sh.at[didx_v[qi]],
                              sem_sc[qr]).wait()

    for q in range(nq - 1):
        idx_issue(q, q)
    _copy_rows(zeros_hbm, acc_sh, s, rtb, tail, NS * rtb)
    plsc.subcore_barrier()
    for g in range(k):
        si_wait(g, g)
        gather_issue(g % nq, g % ring)

    def outer(i, carry):
        for b in range(nq):
            g = i * nq + b
            br = b % ring
            qp = (b - 1) % nq
            qpr = (b - 1) % ring
            qk = (b + k) % nq
            qkr = (b + k) % ring

            @pl.when(g >= 1)
            def _():
                scatter_wait(qp, qpr)

            @pl.when(g + nq - 1 < nchunk)
            def _():
                idx_issue(g + nq - 1, qp)

            @pl.when(g + k < nchunk)
            def _():
                si_wait(g + k, qk)
                gather_issue(qk, qkr)

            gather_wait(b, br)
            di_wait(g, b)
            scatter_issue(b, br)
        return carry

    lax.fori_loop(0, nchunk // nq, outer, 0)
    scatter_wait((nchunk - 1) % nq, (nchunk - 1) % ring)
    plsc.subcore_barrier()
    _copy_rows(acc_sh, out_hbm.at[c], s, rtb, tail, NS * rtb)


def _sc_deg(dst, n, d, zeros_nd, chunk=40, ring=5):
    e = dst.shape[0]
    nchunk = e // (NW * chunk)
    mesh = plsc.VectorSubcoreMesh(core_axis_name="c", subcore_axis_name="s")
    ones = jnp.ones((chunk, d), jnp.float32)
    f = functools.partial(
        pl.kernel,
        out_type=jax.ShapeDtypeStruct((NC, n, d), jnp.float32),
        mesh=mesh,
        scratch_types=(
            [pltpu.VMEM((chunk,), jnp.int32)] * ring
            + [pltpu.VMEM((chunk, d), jnp.float32)]
            + [pltpu.SemaphoreType.DMA] * ring
            + [pltpu.VMEM_SHARED((n, d), jnp.float32)]
        ),
    )(functools.partial(_deg_body, n, nchunk, chunk, ring))
    return f(dst, ones, zeros_nd)


def _sc_agg(hp, src, dst, zeros_nd, chunk=40, ring=5):
    n, d = hp.shape
    e = src.shape[0]
    nchunk = e // (NW * chunk)
    mesh = plsc.VectorSubcoreMesh(core_axis_name="c", subcore_axis_name="s")
    f = functools.partial(
        pl.kernel,
        out_type=jax.ShapeDtypeStruct((NC, n, d), jnp.float32),
        mesh=mesh,
        scratch_types=(
            [pltpu.VMEM((ring, chunk, d), jnp.float32)]
            + [pltpu.VMEM((chunk,), jnp.int32)] * (4 * ring)
            + [pltpu.SemaphoreType.DMA] * (6 * ring)
            + [pltpu.VMEM_SHARED((n, d), jnp.float32)]
        ),
    )(functools.partial(_agg_body, n, nchunk, chunk, ring))
    return f(hp, src, dst, zeros_nd)


def _dis_of(cnt_ref):
    cnt = cnt_ref[0] + cnt_ref[1]
    return lax.rsqrt(cnt + 1.0)[:, 0:1]


def _l1_body(x_ref, w_ref, cnt_ref, hp_ref):
    dis = _dis_of(cnt_ref)
    hp_ref[...] = jnp.dot(x_ref[...], w_ref[...],
                          preferred_element_type=jnp.float32) * dis


def _l2_body(parts_ref, hp1_ref, cnt_ref, b1_ref, w2_ref, hp2_ref):
    dis = _dis_of(cnt_ref)
    h1 = dis * (parts_ref[0] + parts_ref[1] + hp1_ref[...]) + b1_ref[...]
    h1 = jnp.maximum(h1, 0.0)
    hp2_ref[...] = jnp.dot(h1, w2_ref[...],
                           preferred_element_type=jnp.float32) * dis


def _fin_body(parts_ref, hp2_ref, cnt_ref, b2_ref, seg_ref, wout_ref, bout_ref,
              out_ref):
    g = out_ref.shape[0]
    n = hp2_ref.shape[0]
    dis = _dis_of(cnt_ref)
    h2 = dis * (parts_ref[0] + parts_ref[1] + hp2_ref[...]) + b2_ref[...]
    gids = lax.broadcasted_iota(jnp.int32, (g, n), 0)
    onehot = (gids == seg_ref[...]).astype(jnp.float32)
    sums = jnp.dot(onehot, h2, preferred_element_type=jnp.float32)
    cnts = jnp.sum(onehot, axis=1, keepdims=True)
    pooled = sums / jnp.maximum(cnts, 1.0)
    out_ref[...] = jnp.dot(pooled, wout_ref[...],
                           preferred_element_type=jnp.float32) + bout_ref[...]


def _tc(body, out_shape):
    return pl.pallas_call(body, out_shape=out_shape)


def kernel(x, edge_index, batch_seg, W1, b1, W2, b2, W_out, b_out):
    n, d = x.shape
    chunk = 40
    e = edge_index.shape[1]
    nchunk = e // (NW * chunk)
    src = edge_index[0]
    dst = edge_index[1]
    zeros_nd = jnp.zeros((n, d), jnp.float32)

    cnt_parts = _sc_deg(dst, n, d, zeros_nd, chunk)

    hp1 = _tc(_l1_body, jax.ShapeDtypeStruct((n, d), jnp.float32))(
        x, W1, cnt_parts)
    parts1 = _sc_agg(hp1, src, dst, zeros_nd, chunk)

    hp2 = _tc(_l2_body, jax.ShapeDtypeStruct((n, d), jnp.float32))(
        parts1, hp1, cnt_parts, b1, W2)
    parts2 = _sc_agg(hp2, src, dst, zeros_nd, chunk)

    seg2d = batch_seg.reshape(1, n)
    ng = 64
    out = _tc(_fin_body, jax.ShapeDtypeStruct((ng, W_out.shape[1]), jnp.float32))(
        parts2, hp2, cnt_parts, b2, seg2d, W_out, b_out)
    return out
